# baseline (speedup 1.0000x reference)
import functools

import numpy as _np

import jax
import jax.numpy as jnp
from jax import lax
from jax.experimental import pallas as pl
from jax.experimental.pallas import tpu as pltpu
from jax.experimental.pallas import tpu_sc as plsc

K = 1024
D = 32
N = 2304
TBLK = 256
CHUNK = 8
NPAIR = 32
KSPL = 1
KS = K // KSPL

_NC = 2
_NS = 16
_NW = _NC * _NS
_BPW = N // _NW

_DPAD = 128


def _dist_argmin_kernel(z_ref, cb_ref, idx_ref, cbp_ref, cbt8_ref):
    iota_k = lax.broadcasted_iota(jnp.int32, (CHUNK, KS), 1)

    @pl.when(pl.program_id(0) == 0)
    def _():
        cbp_ref[:, :D] = cb_ref[...]
        cbt = jnp.transpose(cb_ref[...], (1, 0))
        for dd in range(D):
            cbt8_ref[dd] = jnp.broadcast_to(cbt[dd:dd + 1, :], (CHUNK, K))

    for r in range(0, TBLK // CHUNK, NPAIR):
        zcs = [z_ref[pl.ds((r + j) * CHUNK, CHUNK), :] for j in range(NPAIR)]
        best = [None] * NPAIR
        for h in range(KSPL):

            def sq(d):
                row = cbt8_ref[d, :, pl.ds(h * KS, KS)]
                out = []
                for j in range(NPAIR):
                    diff = zcs[j][:, d:d + 1] - row
                    out.append(diff * diff)
                return out

            def vadd(a, b):
                return [x + y for x, y in zip(a, b)]

            def grp(g):
                b = 8 * g
                p04 = vadd(sq(b + 0), sq(b + 4))
                p26 = vadd(sq(b + 2), sq(b + 6))
                p15 = vadd(sq(b + 1), sq(b + 5))
                p37 = vadd(sq(b + 3), sq(b + 7))
                return vadd(vadd(p04, p26), vadd(p15, p37))

            dist = vadd(vadd(vadd(grp(0), grp(1)), grp(2)), grp(3))
            for j in range(NPAIR):
                dmin = jnp.min(dist[j], axis=1, keepdims=True)
                cand = jnp.where(dist[j] == dmin, iota_k + h * KS,
                                 jnp.int32(K))
                idx = jnp.min(cand, axis=1, keepdims=True)
                if best[j] is None:
                    best[j] = (dmin, idx)
                else:
                    bv, bi = best[j]
                    take = dmin < bv
                    best[j] = (jnp.minimum(bv, dmin),
                               jnp.where(take, idx, bi))
        for j in range(NPAIR):
            idx_ref[pl.ds((r + j) * CHUNK, CHUNK), :] = best[j][1]


def _argmin_and_pad(z_flat, codebook):
    return pl.pallas_call(
        _dist_argmin_kernel,
        grid=(N // TBLK,),
        in_specs=[
            pl.BlockSpec((TBLK, D), lambda i: (i, 0)),
            pl.BlockSpec((K, D), lambda i: (0, 0)),
        ],
        scratch_shapes=[pltpu.VMEM((D, CHUNK, K), jnp.float32)],
        out_specs=[
            pl.BlockSpec((TBLK, 1), lambda i: (i, 0)),
            pl.BlockSpec((K, _DPAD), lambda i: (0, 0)),
        ],
        out_shape=[
            jax.ShapeDtypeStruct((N, 1), jnp.int32),
            jax.ShapeDtypeStruct((K, _DPAD), jnp.float32),
        ],
    )(z_flat, codebook)


_NTOK = 80
_WPS = D * _NTOK
_PLANE = 32 * 128
_ZQPAD = 4 * 24 * _PLANE
_TRASH = 127

def _soff_table():
    w, d, s = _np.meshgrid(_np.arange(32), _np.arange(D), _np.arange(_NTOK),
                           indexing="ij")
    t = w * 72 + s
    F = t * D + d
    off = (((F // 18432) * 24 + F % 24) * 32 + (F // 576) % 32) * 128 \
        + (F // 24) % 24
    return _np.where(s < 72, off, _TRASH).astype(_np.int32).reshape(-1)


_SOFF = _soff_table()


@functools.cache
def _make_sc_gather():
    @functools.partial(
        pl.kernel,
        mesh=plsc.VectorSubcoreMesh(core_axis_name="c", subcore_axis_name="s"),
        out_type=jax.ShapeDtypeStruct((_ZQPAD,), jnp.float32),
        scratch_types=[
            pltpu.VMEM((_NTOK,), jnp.int32),
            pltpu.VMEM((_WPS,), jnp.int32),
            pltpu.VMEM((_WPS,), jnp.int32),
            pltpu.VMEM((_WPS,), jnp.float32),
            pltpu.SemaphoreType.DMA,
        ],
    )
    def _sc_gather(cbf_hbm, idx_hbm, soff_hbm, out_hbm,
                   idx_v, goff_v, soff_v, vals_v, sem):
        wid = lax.axis_index("s") * _NC + lax.axis_index("c")
        base = wid * _BPW
        zeros = jnp.zeros((16,), jnp.int32)
        idx_v[pl.ds(64, 16)] = zeros
        pltpu.sync_copy(idx_hbm.at[pl.ds(base, _BPW)],
                        idx_v.at[pl.ds(0, _BPW)])
        pltpu.sync_copy(soff_hbm.at[pl.ds(wid * _WPS, _WPS)], soff_v)
        for d in range(D):
            for u in range(_NTOK // 16):
                g = idx_v[pl.ds(u * 16, 16)]
                goff_v[pl.ds(d * _NTOK + u * 16, 16)] = g * 128 + d
        pltpu.async_copy(cbf_hbm.at[goff_v], vals_v, sem).wait()
        pltpu.async_copy(vals_v, out_hbm.at[soff_v], sem).wait()

    return _sc_gather


def kernel(z_e, codebook):
    B, C, H, W = z_e.shape
    z_flat = jnp.transpose(z_e, (0, 2, 3, 1)).reshape(-1, C)
    idx2d, cb_pad = _argmin_and_pad(z_flat, codebook)
    indices = idx2d.reshape(-1)
    zq_pad = _make_sc_gather()(cb_pad.reshape(-1), indices,
                               jnp.asarray(_SOFF))
    z_q = zq_pad.reshape(B, W, C, _DPAD)[..., :H]
    return (z_q, indices)


# speedup vs baseline: 21.2504x; 21.2504x over previous
import functools

import jax
import jax.numpy as jnp
from jax import lax
from jax.experimental import pallas as pl
from jax.experimental.pallas import tpu as pltpu
from jax.experimental.pallas import tpu_sc as plsc

K = 1024
D = 32
N = 2304
TBLK = 256
CHUNK = 8
NPAIR = 32
KSPL = 1
KS = K // KSPL

_NC = 2
_NS = 16
_NW = _NC * _NS
_BPW = N // _NW

_DPAD = 128


def _dist_argmin_kernel(z_ref, cb_ref, idx_ref, cbp_ref, cbt8_ref):
    iota_k = lax.broadcasted_iota(jnp.int32, (CHUNK, KS), 1)

    @pl.when(pl.program_id(0) == 0)
    def _():
        cbp_ref[:, :D] = cb_ref[...]
        cbt = jnp.transpose(cb_ref[...], (1, 0))
        for dd in range(D):
            cbt8_ref[dd] = jnp.broadcast_to(cbt[dd:dd + 1, :], (CHUNK, K))

    for r in range(0, TBLK // CHUNK, NPAIR):
        zcs = [z_ref[pl.ds((r + j) * CHUNK, CHUNK), :] for j in range(NPAIR)]
        best = [None] * NPAIR
        for h in range(KSPL):

            def sq(d):
                row = cbt8_ref[d, :, pl.ds(h * KS, KS)]
                out = []
                for j in range(NPAIR):
                    diff = zcs[j][:, d:d + 1] - row
                    out.append(diff * diff)
                return out

            def vadd(a, b):
                return [x + y for x, y in zip(a, b)]

            def grp(g):
                b = 8 * g
                p04 = vadd(sq(b + 0), sq(b + 4))
                p26 = vadd(sq(b + 2), sq(b + 6))
                p15 = vadd(sq(b + 1), sq(b + 5))
                p37 = vadd(sq(b + 3), sq(b + 7))
                return vadd(vadd(p04, p26), vadd(p15, p37))

            dist = vadd(vadd(vadd(grp(0), grp(1)), grp(2)), grp(3))
            for j in range(NPAIR):
                dmin = jnp.min(dist[j], axis=1, keepdims=True)
                cand = jnp.where(dist[j] == dmin, iota_k + h * KS,
                                 jnp.int32(K))
                idx = jnp.min(cand, axis=1, keepdims=True)
                if best[j] is None:
                    best[j] = (dmin, idx)
                else:
                    bv, bi = best[j]
                    take = dmin < bv
                    best[j] = (jnp.minimum(bv, dmin),
                               jnp.where(take, idx, bi))
        for j in range(NPAIR):
            idx_ref[pl.ds((r + j) * CHUNK, CHUNK), :] = best[j][1]


def _argmin_and_pad(z_flat, codebook):
    return pl.pallas_call(
        _dist_argmin_kernel,
        grid=(N // TBLK,),
        in_specs=[
            pl.BlockSpec((TBLK, D), lambda i: (i, 0)),
            pl.BlockSpec((K, D), lambda i: (0, 0)),
        ],
        scratch_shapes=[pltpu.VMEM((D, CHUNK, K), jnp.float32)],
        out_specs=[
            pl.BlockSpec((TBLK, 1), lambda i: (i, 0)),
            pl.BlockSpec((K, _DPAD), lambda i: (0, 0)),
        ],
        out_shape=[
            jax.ShapeDtypeStruct((N, 1), jnp.int32),
            jax.ShapeDtypeStruct((K, _DPAD), jnp.float32),
        ],
    )(z_flat, codebook)


@functools.cache
def _make_sc_gather():
    @functools.partial(
        pl.kernel,
        mesh=plsc.VectorSubcoreMesh(core_axis_name="c", subcore_axis_name="s"),
        out_type=jax.ShapeDtypeStruct((N, _DPAD), jnp.float32),
        scratch_types=[
            pltpu.VMEM((_BPW,), jnp.int32),
            pltpu.VMEM((_BPW, _DPAD), jnp.float32),
            pltpu.SemaphoreType.DMA,
        ],
    )
    def _sc_gather(cb_hbm, idx_hbm, out_hbm, idx_v, rows_v, sem):
        wid = lax.axis_index("s") * _NC + lax.axis_index("c")
        base = wid * _BPW
        pltpu.sync_copy(idx_hbm.at[pl.ds(base, _BPW)], idx_v)
        pltpu.async_copy(cb_hbm.at[idx_v], rows_v, sem).wait()
        pltpu.sync_copy(rows_v, out_hbm.at[pl.ds(base, _BPW)])

    return _sc_gather


def kernel(z_e, codebook):
    B, C, H, W = z_e.shape
    z_flat = jnp.transpose(z_e, (0, 2, 3, 1)).reshape(-1, C)
    idx2d, cb_pad = _argmin_and_pad(z_flat, codebook)
    indices = idx2d.reshape(-1)
    zq_flat = _make_sc_gather()(cb_pad, indices)[:, :D]
    z_q = jnp.transpose(zq_flat.reshape(B, C, H, W), (0, 3, 1, 2))
    return (z_q, indices)
